# Initial kernel scaffold; baseline (speedup 1.0000x reference)
import jax
import jax.numpy as jnp
from jax.experimental import pallas as pl


def kernel(x, target):
    raise NotImplementedError("write your pallas kernel here")


# speedup vs baseline: 16.4225x; 16.4225x over previous
import functools
import math

import jax
import jax.numpy as jnp
from jax.experimental import pallas as pl
from jax.experimental.pallas import tpu as pltpu

_SIZE = 8192
_PAD = 0
_SMOOTHING = 0.1
_CONF = 1.0 - _SMOOTHING
_EPS = _SMOOTHING / (_SIZE - 2)
_C = (_SIZE - 2) * _EPS * math.log(_EPS) + _CONF * math.log(_CONF)

_ROWS_PER_BLOCK = 256


def _loss_block(x_ref, t_ref, out_ref):
    i = pl.program_id(0)

    @pl.when(i == 0)
    def _():
        out_ref[0, 0] = 0.0

    xb = x_ref[...]
    t = t_ref[0]
    rowsum = jnp.sum(xb, axis=1, keepdims=True)
    x0 = xb[:, 0:1]
    rb, v = xb.shape
    vocab_ids = jax.lax.broadcasted_iota(jnp.int32, (rb, v), 1)
    xt = jnp.sum(jnp.where(vocab_ids == t, xb, 0.0), axis=1, keepdims=True)
    contrib = jnp.where(
        t != _PAD,
        _C - _EPS * rowsum + _EPS * x0 - (_CONF - _EPS) * xt,
        0.0,
    )
    out_ref[0, 0] += jnp.sum(contrib)


def kernel(x, target):
    B, S, V = x.shape
    rows = B * S
    rb = _ROWS_PER_BLOCK
    nblk = rows // rb
    x2 = x.reshape(rows, V)
    t3 = target.reshape(nblk, rb, 1).astype(jnp.int32)
    out = pl.pallas_call(
        _loss_block,
        grid=(nblk,),
        in_specs=[
            pl.BlockSpec((rb, V), lambda i: (i, 0)),
            pl.BlockSpec((1, rb, 1), lambda i: (i, 0, 0)),
        ],
        out_specs=pl.BlockSpec(
            (1, 1), lambda i: (0, 0), memory_space=pltpu.SMEM
        ),
        out_shape=jax.ShapeDtypeStruct((1, 1), jnp.float32),
    )(x2, t3)
    return out[0, 0]
